# initial kernel scaffold (unmeasured)
import jax
import jax.numpy as jnp
from jax import lax
from jax.experimental import pallas as pl
from jax.experimental.pallas import tpu as pltpu

N_Z = 4
M = 2048
D = 2048
CHUNK = M // N_Z


def kernel(partial, resid, gamma):
    x = partial.reshape(M, D)
    g = gamma.reshape(1, D)

    def body(x_ref, resid_ref, g_ref, out_ref, comm_ref,
             send_sems, recv_sems, credit_sem):
        my_x = lax.axis_index("x")
        my_y = lax.axis_index("y")
        my_z = lax.axis_index("z")
        right = (my_z + 1) % N_Z
        left = (my_z - 1) % N_Z

        out_ref[:, :] = x_ref[:, :]

        n_steps = 2 * (N_Z - 1)
        for s in range(n_steps):
            slot = s % 2
            if s < N_Z - 1:
                send_idx = (my_z - s) % N_Z
                recv_idx = (my_z - s - 1) % N_Z
            else:
                t = s - (N_Z - 1)
                send_idx = (my_z + 1 - t) % N_Z
                recv_idx = (my_z - t) % N_Z
            send_off = send_idx * CHUNK
            recv_off = recv_idx * CHUNK

            if s >= 2:
                pl.semaphore_wait(credit_sem, 1)

            rdma = pltpu.make_async_remote_copy(
                src_ref=out_ref.at[pl.ds(send_off, CHUNK), :],
                dst_ref=comm_ref.at[slot],
                send_sem=send_sems.at[slot],
                recv_sem=recv_sems.at[slot],
                device_id=(my_x, my_y, right),
                device_id_type=pl.DeviceIdType.MESH,
            )
            rdma.start()
            rdma.wait()

            if s < N_Z - 1:
                out_ref[pl.ds(recv_off, CHUNK), :] = (
                    out_ref[pl.ds(recv_off, CHUNK), :] + comm_ref[slot]
                )
            else:
                out_ref[pl.ds(recv_off, CHUNK), :] = comm_ref[slot]

            if s + 2 < n_steps:
                pl.semaphore_signal(
                    credit_sem, inc=1,
                    device_id=(my_x, my_y, left),
                    device_id_type=pl.DeviceIdType.MESH,
                )

        y = out_ref[:, :] + resid_ref[:, :]
        ms = jnp.mean(y * y, axis=1, keepdims=True)
        inv = lax.rsqrt(ms + 1e-6)
        out_ref[:, :] = y * inv * g_ref[:, :]

    return pl.pallas_call(
        body,
        out_shape=jax.ShapeDtypeStruct((M, D), jnp.float32),
        in_specs=[
            pl.BlockSpec(memory_space=pltpu.VMEM),
            pl.BlockSpec(memory_space=pltpu.VMEM),
            pl.BlockSpec(memory_space=pltpu.VMEM),
        ],
        out_specs=pl.BlockSpec(memory_space=pltpu.VMEM),
        scratch_shapes=[
            pltpu.VMEM((2, CHUNK, D), jnp.float32),
            pltpu.SemaphoreType.DMA((2,)),
            pltpu.SemaphoreType.DMA((2,)),
            pltpu.SemaphoreType.REGULAR,
        ],
        compiler_params=pltpu.CompilerParams(collective_id=0),
    )(x, resid, g)


# baseline (device time: 332770 ns/iter reference)
import jax
import jax.numpy as jnp
from jax import lax
from jax.experimental import pallas as pl
from jax.experimental.pallas import tpu as pltpu

N_Z = 4
M = 2048
D = 2048
CHUNK = M // N_Z


def kernel(partial, resid, gamma):
    x = partial.reshape(M, D)
    g = gamma.reshape(1, D)

    def body(x_hbm, resid_hbm, g_ref, out_ref, comm_ref, stage_ref,
             send_sems, recv_sems, credit_sem, local_sem):
        my_x = lax.axis_index("x")
        my_y = lax.axis_index("y")
        my_z = lax.axis_index("z")
        right = (my_z + 1) % N_Z
        left = (my_z - 1) % N_Z

        cp = pltpu.make_async_copy(x_hbm, out_ref, local_sem)
        cp.start()
        cp.wait()

        n_steps = 2 * (N_Z - 1)
        for s in range(n_steps):
            slot = s % 2
            if s < N_Z - 1:
                send_idx = (my_z - s) % N_Z
                recv_idx = (my_z - s - 1) % N_Z
            else:
                t = s - (N_Z - 1)
                send_idx = (my_z + 1 - t) % N_Z
                recv_idx = (my_z - t) % N_Z
            send_off = send_idx * CHUNK
            recv_off = recv_idx * CHUNK

            if s >= 2:
                pl.semaphore_wait(credit_sem, 1)

            rdma = pltpu.make_async_remote_copy(
                src_ref=out_ref.at[pl.ds(send_off, CHUNK), :],
                dst_ref=comm_ref.at[slot],
                send_sem=send_sems.at[slot],
                recv_sem=recv_sems.at[slot],
                device_id=(my_x, my_y, right),
                device_id_type=pl.DeviceIdType.MESH,
            )
            rdma.start()
            rdma.wait()

            if s < N_Z - 1:
                out_ref[pl.ds(recv_off, CHUNK), :] = (
                    out_ref[pl.ds(recv_off, CHUNK), :] + comm_ref[slot]
                )
            else:
                out_ref[pl.ds(recv_off, CHUNK), :] = comm_ref[slot]

            if s + 2 < n_steps:
                pl.semaphore_signal(
                    credit_sem, inc=1,
                    device_id=(my_x, my_y, left),
                    device_id_type=pl.DeviceIdType.MESH,
                )

        for c in range(N_Z):
            blk = pl.ds(c * CHUNK, CHUNK)
            cp = pltpu.make_async_copy(
                resid_hbm.at[blk, :], stage_ref, local_sem
            )
            cp.start()
            cp.wait()
            y = out_ref[blk, :] + stage_ref[:, :]
            ms = jnp.mean(y * y, axis=1, keepdims=True)
            inv = lax.rsqrt(ms + 1e-6)
            out_ref[blk, :] = y * inv * g_ref[:, :]

    return pl.pallas_call(
        body,
        out_shape=jax.ShapeDtypeStruct((M, D), jnp.float32),
        in_specs=[
            pl.BlockSpec(memory_space=pl.ANY),
            pl.BlockSpec(memory_space=pl.ANY),
            pl.BlockSpec(memory_space=pltpu.VMEM),
        ],
        out_specs=pl.BlockSpec(memory_space=pltpu.VMEM),
        scratch_shapes=[
            pltpu.VMEM((2, CHUNK, D), jnp.float32),
            pltpu.VMEM((CHUNK, D), jnp.float32),
            pltpu.SemaphoreType.DMA((2,)),
            pltpu.SemaphoreType.DMA((2,)),
            pltpu.SemaphoreType.REGULAR,
            pltpu.SemaphoreType.DMA,
        ],
        compiler_params=pltpu.CompilerParams(
            vmem_limit_bytes=100 * 1024 * 1024,
        ),
    )(x, resid, g)


# device time: 288454 ns/iter; 1.1536x vs baseline; 1.1536x over previous
import jax
import jax.numpy as jnp
from jax import lax
from jax.experimental import pallas as pl
from jax.experimental.pallas import tpu as pltpu

N_Z = 4
N_RING = 8
M = 2048
D = 2048
PIECE = M // N_RING
CHUNK = M // N_Z


def _ring_coords(p):
    px = (p >= 4).astype(jnp.int32)
    py = jnp.where(p < 4, p, 7 - p)
    return px, py


def kernel(partial, resid, gamma):
    x = partial.reshape(M, D)
    g = gamma.reshape(1, D)

    def body(x_hbm, resid_hbm, g_ref, out_ref,
             acc_ref, a_comm, b_comm, stage_ref,
             a_send, a_recv, b_send, b_recv, b_credit, local_sem):
        my_x = lax.axis_index("x")
        my_y = lax.axis_index("y")
        my_z = lax.axis_index("z")

        pos = jnp.where(my_x == 0, my_y, 7 - my_y)
        right_x, right_y = _ring_coords((pos + 1) % N_RING)
        left_x, left_y = _ring_coords((pos + 7) % N_RING)

        my_off = pos * PIECE

        cp = pltpu.make_async_copy(
            x_hbm.at[pl.ds(my_off, PIECE), :], acc_ref, local_sem
        )
        cp.start()
        cp.wait()

        is_edge = jnp.logical_or(my_z == 0, my_z == 3)
        partner = my_z ^ 1
        other_mid = 3 - my_z

        @pl.when(is_edge)
        def _():
            rd = pltpu.make_async_remote_copy(
                src_ref=acc_ref,
                dst_ref=a_comm.at[0],
                send_sem=a_send.at[0],
                recv_sem=a_recv.at[0],
                device_id=(my_x, my_y, partner),
                device_id_type=pl.DeviceIdType.MESH,
            )
            rd.start()
            rd.wait_send()
            fin = pltpu.make_async_remote_copy(
                src_ref=acc_ref,
                dst_ref=a_comm.at[1],
                send_sem=a_send.at[1],
                recv_sem=a_recv.at[1],
                device_id=(my_x, my_y, partner),
                device_id_type=pl.DeviceIdType.MESH,
            )
            fin.wait_recv()
            out_ref[pl.ds(my_off, PIECE), :] = a_comm[1]

        @pl.when(jnp.logical_not(is_edge))
        def _():
            rcv = pltpu.make_async_remote_copy(
                src_ref=acc_ref,
                dst_ref=a_comm.at[0],
                send_sem=a_send.at[1],
                recv_sem=a_recv.at[0],
                device_id=(my_x, my_y, partner),
                device_id_type=pl.DeviceIdType.MESH,
            )
            rcv.wait_recv()
            acc_ref[:, :] = acc_ref[:, :] + a_comm[0]
            ex = pltpu.make_async_remote_copy(
                src_ref=acc_ref,
                dst_ref=a_comm.at[1],
                send_sem=a_send.at[0],
                recv_sem=a_recv.at[1],
                device_id=(my_x, my_y, other_mid),
                device_id_type=pl.DeviceIdType.MESH,
            )
            ex.start()
            ex.wait()
            out_ref[pl.ds(my_off, PIECE), :] = acc_ref[:, :] + a_comm[1]
            fin = pltpu.make_async_remote_copy(
                src_ref=out_ref.at[pl.ds(my_off, PIECE), :],
                dst_ref=a_comm.at[1],
                send_sem=a_send.at[1],
                recv_sem=a_recv.at[1],
                device_id=(my_x, my_y, partner),
                device_id_type=pl.DeviceIdType.MESH,
            )
            fin.start()
            fin.wait_send()

        for h in range(N_RING - 1):
            slot = h % 2
            send_idx = (pos - h) % N_RING
            recv_idx = (pos - h - 1) % N_RING

            if h >= 2:
                pl.semaphore_wait(b_credit, 1)

            rdma = pltpu.make_async_remote_copy(
                src_ref=out_ref.at[pl.ds(send_idx * PIECE, PIECE), :],
                dst_ref=b_comm.at[slot],
                send_sem=b_send.at[slot],
                recv_sem=b_recv.at[slot],
                device_id=(right_x, right_y, my_z),
                device_id_type=pl.DeviceIdType.MESH,
            )
            rdma.start()
            rdma.wait()

            out_ref[pl.ds(recv_idx * PIECE, PIECE), :] = b_comm[slot]

            if h + 2 < N_RING - 1:
                pl.semaphore_signal(
                    b_credit, inc=1,
                    device_id=(left_x, left_y, my_z),
                    device_id_type=pl.DeviceIdType.MESH,
                )

        for c in range(N_Z):
            blk = pl.ds(c * CHUNK, CHUNK)
            cp2 = pltpu.make_async_copy(
                resid_hbm.at[blk, :], stage_ref, local_sem
            )
            cp2.start()
            cp2.wait()
            y = out_ref[blk, :] + stage_ref[:, :]
            ms = jnp.mean(y * y, axis=1, keepdims=True)
            inv = lax.rsqrt(ms + 1e-6)
            out_ref[blk, :] = y * inv * g_ref[:, :]

    return pl.pallas_call(
        body,
        out_shape=jax.ShapeDtypeStruct((M, D), jnp.float32),
        in_specs=[
            pl.BlockSpec(memory_space=pl.ANY),
            pl.BlockSpec(memory_space=pl.ANY),
            pl.BlockSpec(memory_space=pltpu.VMEM),
        ],
        out_specs=pl.BlockSpec(memory_space=pltpu.VMEM),
        scratch_shapes=[
            pltpu.VMEM((PIECE, D), jnp.float32),
            pltpu.VMEM((2, PIECE, D), jnp.float32),
            pltpu.VMEM((2, PIECE, D), jnp.float32),
            pltpu.VMEM((CHUNK, D), jnp.float32),
            pltpu.SemaphoreType.DMA((2,)),
            pltpu.SemaphoreType.DMA((2,)),
            pltpu.SemaphoreType.DMA((2,)),
            pltpu.SemaphoreType.DMA((2,)),
            pltpu.SemaphoreType.REGULAR,
            pltpu.SemaphoreType.DMA,
        ],
        compiler_params=pltpu.CompilerParams(
            vmem_limit_bytes=100 * 1024 * 1024,
        ),
    )(x, resid, g)


# device time: 207875 ns/iter; 1.6008x vs baseline; 1.3876x over previous
import jax
import jax.numpy as jnp
from jax import lax
from jax.experimental import pallas as pl
from jax.experimental.pallas import tpu as pltpu

N_Z = 4
N_RING = 8
M = 2048
D = 2048
PIECE = M // N_RING
N_CW = 4
N_CCW = 3


def _ring_coords(p):
    px = (p >= 4).astype(jnp.int32)
    py = jnp.where(p < 4, p, 7 - p)
    return px, py


def kernel(partial, resid, gamma):
    x = partial.reshape(M, D)
    g = gamma.reshape(1, D)

    def body(x_hbm, resid_hbm, g_ref, out_ref,
             acc_ref, a_comm, b_comm, c_comm, stage_ref,
             a_send, a_recv, b_send, b_recv, c_send, c_recv,
             b_credit, c_credit, local_sem):
        my_x = lax.axis_index("x")
        my_y = lax.axis_index("y")
        my_z = lax.axis_index("z")

        pos = jnp.where(my_x == 0, my_y, 7 - my_y)
        right_x, right_y = _ring_coords((pos + 1) % N_RING)
        left_x, left_y = _ring_coords((pos + 7) % N_RING)

        my_off = pos * PIECE

        cp = pltpu.make_async_copy(
            x_hbm.at[pl.ds(my_off, PIECE), :], acc_ref, local_sem
        )
        cp.start()
        cp.wait()

        is_edge = jnp.logical_or(my_z == 0, my_z == 3)
        partner = my_z ^ 1
        other_mid = 3 - my_z

        @pl.when(is_edge)
        def _():
            rd = pltpu.make_async_remote_copy(
                src_ref=acc_ref,
                dst_ref=a_comm.at[0],
                send_sem=a_send.at[0],
                recv_sem=a_recv.at[0],
                device_id=(my_x, my_y, partner),
                device_id_type=pl.DeviceIdType.MESH,
            )
            rd.start()
            rd.wait_send()
            fin = pltpu.make_async_remote_copy(
                src_ref=acc_ref,
                dst_ref=a_comm.at[1],
                send_sem=a_send.at[1],
                recv_sem=a_recv.at[1],
                device_id=(my_x, my_y, partner),
                device_id_type=pl.DeviceIdType.MESH,
            )
            fin.wait_recv()
            out_ref[pl.ds(my_off, PIECE), :] = a_comm[1]

        @pl.when(jnp.logical_not(is_edge))
        def _():
            rcv = pltpu.make_async_remote_copy(
                src_ref=acc_ref,
                dst_ref=a_comm.at[0],
                send_sem=a_send.at[1],
                recv_sem=a_recv.at[0],
                device_id=(my_x, my_y, partner),
                device_id_type=pl.DeviceIdType.MESH,
            )
            rcv.wait_recv()
            acc_ref[:, :] = acc_ref[:, :] + a_comm[0]
            ex = pltpu.make_async_remote_copy(
                src_ref=acc_ref,
                dst_ref=a_comm.at[1],
                send_sem=a_send.at[0],
                recv_sem=a_recv.at[1],
                device_id=(my_x, my_y, other_mid),
                device_id_type=pl.DeviceIdType.MESH,
            )
            ex.start()
            ex.wait()
            out_ref[pl.ds(my_off, PIECE), :] = acc_ref[:, :] + a_comm[1]
            fin = pltpu.make_async_remote_copy(
                src_ref=out_ref.at[pl.ds(my_off, PIECE), :],
                dst_ref=a_comm.at[1],
                send_sem=a_send.at[1],
                recv_sem=a_recv.at[1],
                device_id=(my_x, my_y, partner),
                device_id_type=pl.DeviceIdType.MESH,
            )
            fin.start()
            fin.wait_send()

        ln_ctr = [0]

        def ln_piece(idx, src_block):
            st = stage_ref.at[ln_ctr[0] % 2]
            ln_ctr[0] += 1
            dma = pltpu.make_async_copy(
                resid_hbm.at[pl.ds(idx * PIECE, PIECE), :], st, local_sem
            )
            dma.start()
            dma.wait()
            y = src_block + st[:, :]
            ms = jnp.mean(y * y, axis=1, keepdims=True)
            inv = lax.rsqrt(ms + 1e-6)
            out_ref[pl.ds(idx * PIECE, PIECE), :] = y * inv * g_ref[:, :]

        def mk_cw(i, src):
            return pltpu.make_async_remote_copy(
                src_ref=src,
                dst_ref=b_comm.at[i % 2],
                send_sem=b_send.at[i % 2],
                recv_sem=b_recv.at[i % 2],
                device_id=(right_x, right_y, my_z),
                device_id_type=pl.DeviceIdType.MESH,
            )

        def mk_ccw(i, src):
            return pltpu.make_async_remote_copy(
                src_ref=src,
                dst_ref=c_comm.at[i % 2],
                send_sem=c_send.at[i % 2],
                recv_sem=c_recv.at[i % 2],
                device_id=(left_x, left_y, my_z),
                device_id_type=pl.DeviceIdType.MESH,
            )

        own_src = out_ref.at[pl.ds(my_off, PIECE), :]
        cw = [None] * N_CW
        ccw = [None] * N_CCW
        cw[0] = mk_cw(0, own_src)
        ccw[0] = mk_ccw(0, own_src)
        cw[0].start()
        ccw[0].start()

        for h in range(N_CW):
            cw[h].wait_recv()
            if h < N_CCW:
                ccw[h].wait_recv()
            cw[h].wait_send()
            if h < N_CCW:
                ccw[h].wait_send()

            if 1 <= h <= N_CW - 2:
                pl.semaphore_signal(
                    b_credit, inc=1,
                    device_id=(left_x, left_y, my_z),
                    device_id_type=pl.DeviceIdType.MESH,
                )
            if 1 <= h <= N_CCW - 2:
                pl.semaphore_signal(
                    c_credit, inc=1,
                    device_id=(right_x, right_y, my_z),
                    device_id_type=pl.DeviceIdType.MESH,
                )

            if h + 1 < N_CW:
                if h + 1 >= 2:
                    pl.semaphore_wait(b_credit, 1)
                cw[h + 1] = mk_cw(h + 1, b_comm.at[h % 2])
                cw[h + 1].start()
            if h + 1 < N_CCW:
                if h + 1 >= 2:
                    pl.semaphore_wait(c_credit, 1)
                ccw[h + 1] = mk_ccw(h + 1, c_comm.at[h % 2])
                ccw[h + 1].start()

            ln_piece((pos - h - 1) % N_RING, b_comm[h % 2])
            if h < N_CCW:
                ln_piece((pos + h + 1) % N_RING, c_comm[h % 2])
            if h == 0:
                ln_piece(pos, out_ref[pl.ds(my_off, PIECE), :])

    return pl.pallas_call(
        body,
        out_shape=jax.ShapeDtypeStruct((M, D), jnp.float32),
        in_specs=[
            pl.BlockSpec(memory_space=pl.ANY),
            pl.BlockSpec(memory_space=pl.ANY),
            pl.BlockSpec(memory_space=pltpu.VMEM),
        ],
        out_specs=pl.BlockSpec(memory_space=pltpu.VMEM),
        scratch_shapes=[
            pltpu.VMEM((PIECE, D), jnp.float32),
            pltpu.VMEM((2, PIECE, D), jnp.float32),
            pltpu.VMEM((2, PIECE, D), jnp.float32),
            pltpu.VMEM((2, PIECE, D), jnp.float32),
            pltpu.VMEM((2, PIECE, D), jnp.float32),
            pltpu.SemaphoreType.DMA((2,)),
            pltpu.SemaphoreType.DMA((2,)),
            pltpu.SemaphoreType.DMA((2,)),
            pltpu.SemaphoreType.DMA((2,)),
            pltpu.SemaphoreType.DMA((2,)),
            pltpu.SemaphoreType.DMA((2,)),
            pltpu.SemaphoreType.REGULAR,
            pltpu.SemaphoreType.REGULAR,
            pltpu.SemaphoreType.DMA,
        ],
        compiler_params=pltpu.CompilerParams(
            vmem_limit_bytes=100 * 1024 * 1024,
        ),
    )(x, resid, g)


# device time: 185131 ns/iter; 1.7975x vs baseline; 1.1229x over previous
import jax
import jax.numpy as jnp
from jax import lax
from jax.experimental import pallas as pl
from jax.experimental.pallas import tpu as pltpu

N_Z = 4
N_RING = 8
M = 2048
D = 2048
PIECE = M // N_RING
HALF = PIECE // 2
N_CW = 4
N_CCW = 3


def _ring_coords(p):
    px = (p >= 4).astype(jnp.int32)
    py = jnp.where(p < 4, p, 7 - p)
    return px, py


def kernel(partial, resid, gamma):
    x = partial.reshape(M, D)
    g = gamma.reshape(1, D)

    def body(x_hbm, resid_hbm, g_ref, out_ref,
             acc_ref, a_comm, b_comm, c_comm, stage_ref,
             a_send, a_recv, b_send, b_recv, c_send, c_recv,
             b_credit, c_credit, local_sem):
        my_x = lax.axis_index("x")
        my_y = lax.axis_index("y")
        my_z = lax.axis_index("z")

        pos = jnp.where(my_x == 0, my_y, 7 - my_y)
        right_x, right_y = _ring_coords((pos + 1) % N_RING)
        left_x, left_y = _ring_coords((pos + 7) % N_RING)

        my_off = pos * PIECE

        cp = pltpu.make_async_copy(
            x_hbm.at[pl.ds(my_off, PIECE), :], acc_ref, local_sem
        )
        cp.start()
        cp.wait()

        is_edge = jnp.logical_or(my_z == 0, my_z == 3)
        partner = my_z ^ 1
        other_mid = 3 - my_z

        def a_rdma(sub, slot, src, dst_z):
            return pltpu.make_async_remote_copy(
                src_ref=src,
                dst_ref=a_comm.at[sub, slot],
                send_sem=a_send.at[sub, slot],
                recv_sem=a_recv.at[sub, slot],
                device_id=(my_x, my_y, dst_z),
                device_id_type=pl.DeviceIdType.MESH,
            )

        def half(sub, ref=None, base=0):
            r = acc_ref if ref is None else ref
            return r.at[pl.ds(base + sub * HALF, HALF), :]

        @pl.when(is_edge)
        def _():
            rd = [a_rdma(s, 0, half(s), partner) for s in range(2)]
            rd[0].start()
            rd[1].start()
            for s in range(2):
                fin = a_rdma(s, 1, half(s), partner)
                fin.wait_recv()
                out_ref[pl.ds(my_off + s * HALF, HALF), :] = a_comm[s, 1]
            rd[0].wait_send()
            rd[1].wait_send()

        @pl.when(jnp.logical_not(is_edge))
        def _():
            rcv = [a_rdma(s, 0, half(s), partner) for s in range(2)]
            ex = [None, None]
            for s in range(2):
                rcv[s].wait_recv()
                acc_ref[pl.ds(s * HALF, HALF), :] = (
                    acc_ref[pl.ds(s * HALF, HALF), :] + a_comm[s, 0]
                )
                ex[s] = a_rdma(s, 1, half(s), other_mid)
                ex[s].start()
            fin = [None, None]
            for s in range(2):
                ex[s].wait()
                out_ref[pl.ds(my_off + s * HALF, HALF), :] = (
                    acc_ref[pl.ds(s * HALF, HALF), :] + a_comm[s, 1]
                )
                fin[s] = a_rdma(
                    s, 1, half(s, out_ref, my_off), partner
                )
                fin[s].start()
            fin[0].wait_send()
            fin[1].wait_send()

        ln_ctr = [0]

        def ln_piece(idx, src_block):
            st = stage_ref.at[ln_ctr[0] % 2]
            ln_ctr[0] += 1
            dma = pltpu.make_async_copy(
                resid_hbm.at[pl.ds(idx * PIECE, PIECE), :], st, local_sem
            )
            dma.start()
            dma.wait()
            y = src_block + st[:, :]
            ms = jnp.mean(y * y, axis=1, keepdims=True)
            inv = lax.rsqrt(ms + 1e-6)
            out_ref[pl.ds(idx * PIECE, PIECE), :] = y * inv * g_ref[:, :]

        def mk_cw(i, src):
            return pltpu.make_async_remote_copy(
                src_ref=src,
                dst_ref=b_comm.at[i % 2],
                send_sem=b_send.at[i % 2],
                recv_sem=b_recv.at[i % 2],
                device_id=(right_x, right_y, my_z),
                device_id_type=pl.DeviceIdType.MESH,
            )

        def mk_ccw(i, src):
            return pltpu.make_async_remote_copy(
                src_ref=src,
                dst_ref=c_comm.at[i % 2],
                send_sem=c_send.at[i % 2],
                recv_sem=c_recv.at[i % 2],
                device_id=(left_x, left_y, my_z),
                device_id_type=pl.DeviceIdType.MESH,
            )

        own_src = out_ref.at[pl.ds(my_off, PIECE), :]
        cw = [None] * N_CW
        ccw = [None] * N_CCW
        cw[0] = mk_cw(0, own_src)
        ccw[0] = mk_ccw(0, own_src)
        cw[0].start()
        ccw[0].start()

        for h in range(N_CW):
            cw[h].wait_recv()
            if h < N_CCW:
                ccw[h].wait_recv()
            cw[h].wait_send()
            if h < N_CCW:
                ccw[h].wait_send()

            if 1 <= h <= N_CW - 2:
                pl.semaphore_signal(
                    b_credit, inc=1,
                    device_id=(left_x, left_y, my_z),
                    device_id_type=pl.DeviceIdType.MESH,
                )
            if 1 <= h <= N_CCW - 2:
                pl.semaphore_signal(
                    c_credit, inc=1,
                    device_id=(right_x, right_y, my_z),
                    device_id_type=pl.DeviceIdType.MESH,
                )

            if h + 1 < N_CW:
                if h + 1 >= 2:
                    pl.semaphore_wait(b_credit, 1)
                cw[h + 1] = mk_cw(h + 1, b_comm.at[h % 2])
                cw[h + 1].start()
            if h + 1 < N_CCW:
                if h + 1 >= 2:
                    pl.semaphore_wait(c_credit, 1)
                ccw[h + 1] = mk_ccw(h + 1, c_comm.at[h % 2])
                ccw[h + 1].start()

            ln_piece((pos - h - 1) % N_RING, b_comm[h % 2])
            if h < N_CCW:
                ln_piece((pos + h + 1) % N_RING, c_comm[h % 2])
            if h == 0:
                ln_piece(pos, out_ref[pl.ds(my_off, PIECE), :])

    return pl.pallas_call(
        body,
        out_shape=jax.ShapeDtypeStruct((M, D), jnp.float32),
        in_specs=[
            pl.BlockSpec(memory_space=pl.ANY),
            pl.BlockSpec(memory_space=pl.ANY),
            pl.BlockSpec(memory_space=pltpu.VMEM),
        ],
        out_specs=pl.BlockSpec(memory_space=pltpu.VMEM),
        scratch_shapes=[
            pltpu.VMEM((PIECE, D), jnp.float32),
            pltpu.VMEM((2, 2, HALF, D), jnp.float32),
            pltpu.VMEM((2, PIECE, D), jnp.float32),
            pltpu.VMEM((2, PIECE, D), jnp.float32),
            pltpu.VMEM((2, PIECE, D), jnp.float32),
            pltpu.SemaphoreType.DMA((2, 2)),
            pltpu.SemaphoreType.DMA((2, 2)),
            pltpu.SemaphoreType.DMA((2,)),
            pltpu.SemaphoreType.DMA((2,)),
            pltpu.SemaphoreType.DMA((2,)),
            pltpu.SemaphoreType.DMA((2,)),
            pltpu.SemaphoreType.REGULAR,
            pltpu.SemaphoreType.REGULAR,
            pltpu.SemaphoreType.DMA,
        ],
        compiler_params=pltpu.CompilerParams(
            vmem_limit_bytes=100 * 1024 * 1024,
        ),
    )(x, resid, g)


# device time: 165219 ns/iter; 2.0141x vs baseline; 1.1205x over previous
import jax
import jax.numpy as jnp
from jax import lax
from jax.experimental import pallas as pl
from jax.experimental.pallas import tpu as pltpu

N_Z = 4
N_RING = 8
M = 2048
D = 2048
PIECE = M // N_RING
HALF = PIECE // 2
N_HOP = 4


def _ring_coords(p):
    px = (p >= 4).astype(jnp.int32)
    py = jnp.where(p < 4, p, 7 - p)
    return px, py


def kernel(partial, resid, gamma):
    x = partial.reshape(M, D)
    g = gamma.reshape(1, D)

    def body(x_hbm, resid_hbm, g_ref, out_ref,
             acc_ref, a_comm, b_comm, c_comm, stage_ref,
             a_send, a_recv, b_send, b_recv, c_send, c_recv,
             b_credit, c_credit, local_sem, stage_sems):
        my_x = lax.axis_index("x")
        my_y = lax.axis_index("y")
        my_z = lax.axis_index("z")

        pos = jnp.where(my_x == 0, my_y, 7 - my_y)
        right_x, right_y = _ring_coords((pos + 1) % N_RING)
        left_x, left_y = _ring_coords((pos + 7) % N_RING)

        my_off = pos * PIECE

        is_edge = jnp.logical_or(my_z == 0, my_z == 3)
        partner = my_z ^ 1
        other_mid = 3 - my_z

        bar = pltpu.get_barrier_semaphore()
        for did in (
            (my_x, my_y, partner),
            (left_x, left_y, my_z),
            (right_x, right_y, my_z),
        ):
            pl.semaphore_signal(
                bar, inc=1, device_id=did,
                device_id_type=pl.DeviceIdType.MESH,
            )

        @pl.when(jnp.logical_not(is_edge))
        def _():
            pl.semaphore_signal(
                bar, inc=1, device_id=(my_x, my_y, other_mid),
                device_id_type=pl.DeviceIdType.MESH,
            )

        pl.semaphore_wait(bar, 3)

        @pl.when(jnp.logical_not(is_edge))
        def _():
            pl.semaphore_wait(bar, 1)

        cp = pltpu.make_async_copy(
            x_hbm.at[pl.ds(my_off, PIECE), :], acc_ref, local_sem
        )
        cp.start()

        pf_ctr = [0]

        def prefetch_piece(idx):
            slot = pf_ctr[0] % 4
            pf_ctr[0] += 1
            dma = pltpu.make_async_copy(
                resid_hbm.at[pl.ds(idx * PIECE, PIECE), :],
                stage_ref.at[slot],
                stage_sems.at[slot],
            )
            dma.start()
            return (dma, slot)

        pf_m1 = prefetch_piece((pos - 1) % N_RING)
        pf_p1 = prefetch_piece((pos + 1) % N_RING)
        pf_own = prefetch_piece(pos)

        cp.wait()

        def a_rdma(sub, slot, src, dst_z):
            return pltpu.make_async_remote_copy(
                src_ref=src,
                dst_ref=a_comm.at[sub, slot],
                send_sem=a_send.at[sub, slot],
                recv_sem=a_recv.at[sub, slot],
                device_id=(my_x, my_y, dst_z),
                device_id_type=pl.DeviceIdType.MESH,
            )

        def half(sub, ref=None, base=0):
            r = acc_ref if ref is None else ref
            return r.at[pl.ds(base + sub * HALF, HALF), :]

        @pl.when(is_edge)
        def _():
            rd = [a_rdma(s, 0, half(s), partner) for s in range(2)]
            rd[0].start()
            rd[1].start()
            for s in range(2):
                fin = a_rdma(s, 1, half(s), partner)
                fin.wait_recv()
                out_ref[pl.ds(my_off + s * HALF, HALF), :] = a_comm[s, 1]
            rd[0].wait_send()
            rd[1].wait_send()

        @pl.when(jnp.logical_not(is_edge))
        def _():
            rcv = [a_rdma(s, 0, half(s), partner) for s in range(2)]
            ex = [None, None]
            for s in range(2):
                rcv[s].wait_recv()
                acc_ref[pl.ds(s * HALF, HALF), :] = (
                    acc_ref[pl.ds(s * HALF, HALF), :] + a_comm[s, 0]
                )
                ex[s] = a_rdma(s, 1, half(s), other_mid)
                ex[s].start()
            fin = [None, None]
            for s in range(2):
                ex[s].wait()
                out_ref[pl.ds(my_off + s * HALF, HALF), :] = (
                    acc_ref[pl.ds(s * HALF, HALF), :] + a_comm[s, 1]
                )
                fin[s] = a_rdma(
                    s, 1, half(s, out_ref, my_off), partner
                )
                fin[s].start()
            fin[0].wait_send()
            fin[1].wait_send()

        def ln_from(pf, row0, src_block, nrows=PIECE, sub_off=0, wait=True):
            dma, slot = pf
            if wait:
                dma.wait()
            if nrows == PIECE:
                st = stage_ref[slot]
            elif sub_off == 0:
                st = stage_ref[slot, :HALF, :]
            else:
                st = stage_ref[slot, HALF:, :]
            y = src_block + st
            ms = jnp.mean(y * y, axis=1, keepdims=True)
            inv = lax.rsqrt(ms + 1e-6)
            out_ref[pl.ds(row0, nrows), :] = y * inv * g_ref[:, :]

        pf_cw = {0: pf_m1}
        pf_ccw = {0: pf_p1}
        idx4 = (pos + N_HOP) % N_RING
        pf4 = None

        def mk_hop(i, comm, send_sems, recv_sems, tgt, src):
            if i == N_HOP - 1:
                dst = comm.at[i % 2, pl.ds(0, HALF), :] if tgt == "cw" \
                    else comm.at[i % 2, pl.ds(HALF, HALF), :]
            else:
                dst = comm.at[i % 2]
            dev = (right_x, right_y, my_z) if tgt == "cw" \
                else (left_x, left_y, my_z)
            return pltpu.make_async_remote_copy(
                src_ref=src,
                dst_ref=dst,
                send_sem=send_sems.at[i % 2],
                recv_sem=recv_sems.at[i % 2],
                device_id=dev,
                device_id_type=pl.DeviceIdType.MESH,
            )

        def mk_cw(i, src):
            return mk_hop(i, b_comm, b_send, b_recv, "cw", src)

        def mk_ccw(i, src):
            return mk_hop(i, c_comm, c_send, c_recv, "ccw", src)

        own_src = out_ref.at[pl.ds(my_off, PIECE), :]
        cw = [None] * N_HOP
        ccw = [None] * N_HOP
        cw[0] = mk_cw(0, own_src)
        ccw[0] = mk_ccw(0, own_src)
        cw[0].start()
        ccw[0].start()

        for h in range(N_HOP):
            cw[h].wait_recv()
            ccw[h].wait_recv()
            cw[h].wait_send()
            ccw[h].wait_send()

            if 1 <= h <= N_HOP - 2:
                pl.semaphore_signal(
                    b_credit, inc=1,
                    device_id=(left_x, left_y, my_z),
                    device_id_type=pl.DeviceIdType.MESH,
                )
                pl.semaphore_signal(
                    c_credit, inc=1,
                    device_id=(right_x, right_y, my_z),
                    device_id_type=pl.DeviceIdType.MESH,
                )

            if h + 1 < N_HOP:
                if h + 1 >= 2:
                    pl.semaphore_wait(b_credit, 1)
                cw_src = (
                    b_comm.at[h % 2] if h + 1 < N_HOP - 1
                    else b_comm.at[h % 2, pl.ds(0, HALF), :]
                )
                cw[h + 1] = mk_cw(h + 1, cw_src)
                cw[h + 1].start()
                if h + 1 >= 2:
                    pl.semaphore_wait(c_credit, 1)
                ccw_src = (
                    c_comm.at[h % 2] if h + 1 < N_HOP - 1
                    else c_comm.at[h % 2, pl.ds(HALF, HALF), :]
                )
                ccw[h + 1] = mk_ccw(h + 1, ccw_src)
                ccw[h + 1].start()

            if h < N_HOP - 1:
                ln_from(pf_cw[h], ((pos - h - 1) % N_RING) * PIECE,
                        b_comm[h % 2])
                ln_from(pf_ccw[h], ((pos + h + 1) % N_RING) * PIECE,
                        c_comm[h % 2])
                if h == 0:
                    ln_from(pf_own, my_off, out_ref[pl.ds(my_off, PIECE), :])
                if h + 2 < N_HOP:
                    pf_cw[h + 1] = prefetch_piece((pos - h - 2) % N_RING)
                    pf_ccw[h + 1] = prefetch_piece((pos + h + 2) % N_RING)
                else:
                    pf4 = prefetch_piece(idx4)
            else:
                ln_from(pf4, idx4 * PIECE, b_comm[h % 2, :HALF, :],
                        nrows=HALF, sub_off=0)
                ln_from(pf4, idx4 * PIECE + HALF, c_comm[h % 2, HALF:, :],
                        nrows=HALF, sub_off=1, wait=False)

    return pl.pallas_call(
        body,
        out_shape=jax.ShapeDtypeStruct((M, D), jnp.float32),
        in_specs=[
            pl.BlockSpec(memory_space=pl.ANY),
            pl.BlockSpec(memory_space=pl.ANY),
            pl.BlockSpec(memory_space=pltpu.VMEM),
        ],
        out_specs=pl.BlockSpec(memory_space=pltpu.VMEM),
        scratch_shapes=[
            pltpu.VMEM((PIECE, D), jnp.float32),
            pltpu.VMEM((2, 2, HALF, D), jnp.float32),
            pltpu.VMEM((2, PIECE, D), jnp.float32),
            pltpu.VMEM((2, PIECE, D), jnp.float32),
            pltpu.VMEM((4, PIECE, D), jnp.float32),
            pltpu.SemaphoreType.DMA((2, 2)),
            pltpu.SemaphoreType.DMA((2, 2)),
            pltpu.SemaphoreType.DMA((2,)),
            pltpu.SemaphoreType.DMA((2,)),
            pltpu.SemaphoreType.DMA((2,)),
            pltpu.SemaphoreType.DMA((2,)),
            pltpu.SemaphoreType.REGULAR,
            pltpu.SemaphoreType.REGULAR,
            pltpu.SemaphoreType.DMA,
            pltpu.SemaphoreType.DMA((4,)),
        ],
        compiler_params=pltpu.CompilerParams(
            vmem_limit_bytes=100 * 1024 * 1024,
            collective_id=0,
        ),
    )(x, resid, g)


# device time: 154101 ns/iter; 2.1594x vs baseline; 1.0721x over previous
import jax
import jax.numpy as jnp
from jax import lax
from jax.experimental import pallas as pl
from jax.experimental.pallas import tpu as pltpu

N_Z = 4
N_RING = 8
M = 2048
D = 2048
PIECE = M // N_RING
HALF = PIECE // 2
N_HOP = 4


def _ring_coords(p):
    px = (p >= 4).astype(jnp.int32)
    py = jnp.where(p < 4, p, 7 - p)
    return px, py


def kernel(partial, resid, gamma):
    x = partial.reshape(M, D)
    g = gamma.reshape(1, D)

    def body(x_hbm, resid_hbm, g_ref, out_ref,
             acc_ref, a_comm, b_comm, c_comm, stage_ref,
             a_send, a_recv, b_send, b_recv, c_send, c_recv,
             b_credit, c_credit, local_sem, stage_sems):
        my_x = lax.axis_index("x")
        my_y = lax.axis_index("y")
        my_z = lax.axis_index("z")

        pos = jnp.where(my_x == 0, my_y, 7 - my_y)
        right_x, right_y = _ring_coords((pos + 1) % N_RING)
        left_x, left_y = _ring_coords((pos + 7) % N_RING)

        my_off = pos * PIECE

        is_edge = jnp.logical_or(my_z == 0, my_z == 3)
        partner = my_z ^ 1
        other_mid = 3 - my_z

        bar = pltpu.get_barrier_semaphore()
        for did in (
            (my_x, my_y, partner),
            (left_x, left_y, my_z),
            (right_x, right_y, my_z),
        ):
            pl.semaphore_signal(
                bar, inc=1, device_id=did,
                device_id_type=pl.DeviceIdType.MESH,
            )

        @pl.when(jnp.logical_not(is_edge))
        def _():
            pl.semaphore_signal(
                bar, inc=1, device_id=(my_x, my_y, other_mid),
                device_id_type=pl.DeviceIdType.MESH,
            )

        pl.semaphore_wait(bar, 3)

        @pl.when(jnp.logical_not(is_edge))
        def _():
            pl.semaphore_wait(bar, 1)

        cp = pltpu.make_async_copy(
            x_hbm.at[pl.ds(my_off, PIECE), :], acc_ref, local_sem
        )
        cp.start()

        pf_ctr = [0]

        def prefetch_piece(idx):
            slot = pf_ctr[0] % 4
            pf_ctr[0] += 1
            dma = pltpu.make_async_copy(
                resid_hbm.at[pl.ds(idx * PIECE, PIECE), :],
                stage_ref.at[slot],
                stage_sems.at[slot],
            )
            dma.start()
            return (dma, slot)

        pf_m1 = prefetch_piece((pos - 1) % N_RING)
        pf_p1 = prefetch_piece((pos + 1) % N_RING)
        pf_own = prefetch_piece(pos)

        cp.wait()

        def a_rdma(sub, slot, src, dst_z):
            return pltpu.make_async_remote_copy(
                src_ref=src,
                dst_ref=a_comm.at[sub, slot],
                send_sem=a_send.at[sub, slot],
                recv_sem=a_recv.at[sub, slot],
                device_id=(my_x, my_y, dst_z),
                device_id_type=pl.DeviceIdType.MESH,
            )

        def half(sub, ref=None, base=0):
            r = acc_ref if ref is None else ref
            return r.at[pl.ds(base + sub * HALF, HALF), :]

        def mk_hop0(sub, comm, send_sems, recv_sems, dev):
            sem = 0 if sub == 0 else 2
            return pltpu.make_async_remote_copy(
                src_ref=half(sub, out_ref, my_off),
                dst_ref=comm.at[0, pl.ds(sub * HALF, HALF), :],
                send_sem=send_sems.at[sem],
                recv_sem=recv_sems.at[sem],
                device_id=dev,
                device_id_type=pl.DeviceIdType.MESH,
            )

        cw0 = [mk_hop0(s, b_comm, b_send, b_recv,
                       (right_x, right_y, my_z)) for s in range(2)]
        ccw0 = [mk_hop0(s, c_comm, c_send, c_recv,
                        (left_x, left_y, my_z)) for s in range(2)]

        @pl.when(is_edge)
        def _():
            rd = [a_rdma(s, 0, half(s), partner) for s in range(2)]
            rd[0].start()
            rd[1].start()
            for s in range(2):
                fin = a_rdma(s, 1, half(s), partner)
                fin.wait_recv()
                out_ref[pl.ds(my_off + s * HALF, HALF), :] = a_comm[s, 1]
                cw0[s].start()
                ccw0[s].start()
            rd[0].wait_send()
            rd[1].wait_send()

        @pl.when(jnp.logical_not(is_edge))
        def _():
            rcv = [a_rdma(s, 0, half(s), partner) for s in range(2)]
            ex = [None, None]
            for s in range(2):
                rcv[s].wait_recv()
                acc_ref[pl.ds(s * HALF, HALF), :] = (
                    acc_ref[pl.ds(s * HALF, HALF), :] + a_comm[s, 0]
                )
                ex[s] = a_rdma(s, 1, half(s), other_mid)
                ex[s].start()
            fin = [None, None]
            for s in range(2):
                ex[s].wait()
                out_ref[pl.ds(my_off + s * HALF, HALF), :] = (
                    acc_ref[pl.ds(s * HALF, HALF), :] + a_comm[s, 1]
                )
                fin[s] = a_rdma(
                    s, 1, half(s, out_ref, my_off), partner
                )
                fin[s].start()
                cw0[s].start()
                ccw0[s].start()
            fin[0].wait_send()
            fin[1].wait_send()

        def ln_from(pf, row0, src_block, nrows=PIECE, sub_off=0, wait=True):
            dma, slot = pf
            if wait:
                dma.wait()
            if nrows == PIECE:
                st = stage_ref[slot]
            elif sub_off == 0:
                st = stage_ref[slot, :HALF, :]
            else:
                st = stage_ref[slot, HALF:, :]
            y = src_block + st
            ms = jnp.mean(y * y, axis=1, keepdims=True)
            inv = lax.rsqrt(ms + 1e-6)
            out_ref[pl.ds(row0, nrows), :] = y * inv * g_ref[:, :]

        pf_cw = {0: pf_m1}
        pf_ccw = {0: pf_p1}
        idx4 = (pos + N_HOP) % N_RING
        pf4 = None

        def mk_hop(i, comm, send_sems, recv_sems, tgt, src):
            if i == N_HOP - 1:
                dst = comm.at[i % 2, pl.ds(0, HALF), :] if tgt == "cw" \
                    else comm.at[i % 2, pl.ds(HALF, HALF), :]
            else:
                dst = comm.at[i % 2]
            dev = (right_x, right_y, my_z) if tgt == "cw" \
                else (left_x, left_y, my_z)
            return pltpu.make_async_remote_copy(
                src_ref=src,
                dst_ref=dst,
                send_sem=send_sems.at[i % 2],
                recv_sem=recv_sems.at[i % 2],
                device_id=dev,
                device_id_type=pl.DeviceIdType.MESH,
            )

        def mk_cw(i, src):
            return mk_hop(i, b_comm, b_send, b_recv, "cw", src)

        def mk_ccw(i, src):
            return mk_hop(i, c_comm, c_send, c_recv, "ccw", src)

        cw = [cw0] + [None] * (N_HOP - 1)
        ccw = [ccw0] + [None] * (N_HOP - 1)

        for h in range(N_HOP):
            if h == 0:
                for d in (*cw0, *ccw0):
                    d.wait_recv()
                for d in (*cw0, *ccw0):
                    d.wait_send()
            else:
                cw[h].wait_recv()
                ccw[h].wait_recv()
                cw[h].wait_send()
                ccw[h].wait_send()

            if 1 <= h <= N_HOP - 2:
                pl.semaphore_signal(
                    b_credit, inc=1,
                    device_id=(left_x, left_y, my_z),
                    device_id_type=pl.DeviceIdType.MESH,
                )
                pl.semaphore_signal(
                    c_credit, inc=1,
                    device_id=(right_x, right_y, my_z),
                    device_id_type=pl.DeviceIdType.MESH,
                )

            if h + 1 < N_HOP:
                if h + 1 >= 2:
                    pl.semaphore_wait(b_credit, 1)
                cw_src = (
                    b_comm.at[h % 2] if h + 1 < N_HOP - 1
                    else b_comm.at[h % 2, pl.ds(0, HALF), :]
                )
                cw[h + 1] = mk_cw(h + 1, cw_src)
                cw[h + 1].start()
                if h + 1 >= 2:
                    pl.semaphore_wait(c_credit, 1)
                ccw_src = (
                    c_comm.at[h % 2] if h + 1 < N_HOP - 1
                    else c_comm.at[h % 2, pl.ds(HALF, HALF), :]
                )
                ccw[h + 1] = mk_ccw(h + 1, ccw_src)
                ccw[h + 1].start()

            if h < N_HOP - 1:
                ln_from(pf_cw[h], ((pos - h - 1) % N_RING) * PIECE,
                        b_comm[h % 2])
                ln_from(pf_ccw[h], ((pos + h + 1) % N_RING) * PIECE,
                        c_comm[h % 2])
                if h == 0:
                    ln_from(pf_own, my_off, out_ref[pl.ds(my_off, PIECE), :])
                if h + 2 < N_HOP:
                    pf_cw[h + 1] = prefetch_piece((pos - h - 2) % N_RING)
                    pf_ccw[h + 1] = prefetch_piece((pos + h + 2) % N_RING)
                else:
                    pf4 = prefetch_piece(idx4)
            else:
                ln_from(pf4, idx4 * PIECE, b_comm[h % 2, :HALF, :],
                        nrows=HALF, sub_off=0)
                ln_from(pf4, idx4 * PIECE + HALF, c_comm[h % 2, HALF:, :],
                        nrows=HALF, sub_off=1, wait=False)

    return pl.pallas_call(
        body,
        out_shape=jax.ShapeDtypeStruct((M, D), jnp.float32),
        in_specs=[
            pl.BlockSpec(memory_space=pl.ANY),
            pl.BlockSpec(memory_space=pl.ANY),
            pl.BlockSpec(memory_space=pltpu.VMEM),
        ],
        out_specs=pl.BlockSpec(memory_space=pltpu.VMEM),
        scratch_shapes=[
            pltpu.VMEM((PIECE, D), jnp.float32),
            pltpu.VMEM((2, 2, HALF, D), jnp.float32),
            pltpu.VMEM((2, PIECE, D), jnp.float32),
            pltpu.VMEM((2, PIECE, D), jnp.float32),
            pltpu.VMEM((4, PIECE, D), jnp.float32),
            pltpu.SemaphoreType.DMA((2, 2)),
            pltpu.SemaphoreType.DMA((2, 2)),
            pltpu.SemaphoreType.DMA((3,)),
            pltpu.SemaphoreType.DMA((3,)),
            pltpu.SemaphoreType.DMA((3,)),
            pltpu.SemaphoreType.DMA((3,)),
            pltpu.SemaphoreType.REGULAR,
            pltpu.SemaphoreType.REGULAR,
            pltpu.SemaphoreType.DMA,
            pltpu.SemaphoreType.DMA((4,)),
        ],
        compiler_params=pltpu.CompilerParams(
            vmem_limit_bytes=100 * 1024 * 1024,
            collective_id=0,
        ),
    )(x, resid, g)


# device time: 151772 ns/iter; 2.1926x vs baseline; 1.0153x over previous
import jax
import jax.numpy as jnp
from jax import lax
from jax.experimental import pallas as pl
from jax.experimental.pallas import tpu as pltpu

N_Z = 4
N_RING = 8
M = 2048
D = 2048
PIECE = M // N_RING
HALF = PIECE // 2
N_SUB = 4
QTR = PIECE // N_SUB
N_HOP = 4


def _ring_coords(p):
    px = (p >= 4).astype(jnp.int32)
    py = jnp.where(p < 4, p, 7 - p)
    return px, py


def kernel(partial, resid, gamma):
    x = partial.reshape(M, D)
    g = gamma.reshape(1, D)

    def body(x_hbm, resid_hbm, g_ref, out_ref,
             acc_ref, a_comm, b_comm, c_comm, stage_ref,
             a_send, a_recv, b_send, b_recv, c_send, c_recv,
             b_credit, c_credit, local_sem, stage_sems):
        my_x = lax.axis_index("x")
        my_y = lax.axis_index("y")
        my_z = lax.axis_index("z")

        pos = jnp.where(my_x == 0, my_y, 7 - my_y)
        right_x, right_y = _ring_coords((pos + 1) % N_RING)
        left_x, left_y = _ring_coords((pos + 7) % N_RING)

        my_off = pos * PIECE

        is_edge = jnp.logical_or(my_z == 0, my_z == 3)
        partner = my_z ^ 1
        other_mid = 3 - my_z

        bar = pltpu.get_barrier_semaphore()
        for did in (
            (my_x, my_y, partner),
            (left_x, left_y, my_z),
            (right_x, right_y, my_z),
        ):
            pl.semaphore_signal(
                bar, inc=1, device_id=did,
                device_id_type=pl.DeviceIdType.MESH,
            )

        @pl.when(jnp.logical_not(is_edge))
        def _():
            pl.semaphore_signal(
                bar, inc=1, device_id=(my_x, my_y, other_mid),
                device_id_type=pl.DeviceIdType.MESH,
            )

        pl.semaphore_wait(bar, 3)

        @pl.when(jnp.logical_not(is_edge))
        def _():
            pl.semaphore_wait(bar, 1)

        cp = pltpu.make_async_copy(
            x_hbm.at[pl.ds(my_off, PIECE), :], acc_ref, local_sem
        )
        cp.start()

        pf_ctr = [0]

        def prefetch_piece(idx):
            slot = pf_ctr[0] % 4
            pf_ctr[0] += 1
            dma = pltpu.make_async_copy(
                resid_hbm.at[pl.ds(idx * PIECE, PIECE), :],
                stage_ref.at[slot],
                stage_sems.at[slot],
            )
            dma.start()
            return (dma, slot)

        pf_m1 = prefetch_piece((pos - 1) % N_RING)
        pf_p1 = prefetch_piece((pos + 1) % N_RING)
        pf_own = prefetch_piece(pos)

        cp.wait()

        def a_rdma(sub, slot, src, dst_z):
            return pltpu.make_async_remote_copy(
                src_ref=src,
                dst_ref=a_comm.at[sub, slot],
                send_sem=a_send.at[sub, slot],
                recv_sem=a_recv.at[sub, slot],
                device_id=(my_x, my_y, dst_z),
                device_id_type=pl.DeviceIdType.MESH,
            )

        def qtr(sub, ref=None, base=0):
            r = acc_ref if ref is None else ref
            return r.at[pl.ds(base + sub * QTR, QTR), :]

        def half(sub, ref=None, base=0):
            r = acc_ref if ref is None else ref
            return r.at[pl.ds(base + sub * HALF, HALF), :]

        def mk_hop0(sub, comm, send_sems, recv_sems, dev):
            sem = 0 if sub == 0 else 2
            return pltpu.make_async_remote_copy(
                src_ref=half(sub, out_ref, my_off),
                dst_ref=comm.at[0, pl.ds(sub * HALF, HALF), :],
                send_sem=send_sems.at[sem],
                recv_sem=recv_sems.at[sem],
                device_id=dev,
                device_id_type=pl.DeviceIdType.MESH,
            )

        cw0 = [mk_hop0(s, b_comm, b_send, b_recv,
                       (right_x, right_y, my_z)) for s in range(2)]
        ccw0 = [mk_hop0(s, c_comm, c_send, c_recv,
                        (left_x, left_y, my_z)) for s in range(2)]

        @pl.when(is_edge)
        def _():
            rd = [a_rdma(s, 0, qtr(s), partner) for s in range(N_SUB)]
            for s in range(N_SUB):
                rd[s].start()
            for s in range(N_SUB):
                fin = a_rdma(s, 1, qtr(s), partner)
                fin.wait_recv()
                out_ref[pl.ds(my_off + s * QTR, QTR), :] = a_comm[s, 1]
                if s % 2 == 1:
                    cw0[s // 2].start()
                    ccw0[s // 2].start()
            for s in range(N_SUB):
                rd[s].wait_send()

        @pl.when(jnp.logical_not(is_edge))
        def _():
            rcv = [a_rdma(s, 0, qtr(s), partner) for s in range(N_SUB)]
            ex = [None] * N_SUB
            for s in range(N_SUB):
                rcv[s].wait_recv()
                acc_ref[pl.ds(s * QTR, QTR), :] = (
                    acc_ref[pl.ds(s * QTR, QTR), :] + a_comm[s, 0]
                )
                ex[s] = a_rdma(s, 1, qtr(s), other_mid)
                ex[s].start()
            fin = [None] * N_SUB
            for s in range(N_SUB):
                ex[s].wait()
                out_ref[pl.ds(my_off + s * QTR, QTR), :] = (
                    acc_ref[pl.ds(s * QTR, QTR), :] + a_comm[s, 1]
                )
                fin[s] = a_rdma(
                    s, 1, qtr(s, out_ref, my_off), partner
                )
                fin[s].start()
                if s % 2 == 1:
                    cw0[s // 2].start()
                    ccw0[s // 2].start()
            for s in range(N_SUB):
                fin[s].wait_send()

        def ln_from(pf, row0, src_block, nrows=PIECE, sub_off=0, wait=True):
            dma, slot = pf
            if wait:
                dma.wait()
            if nrows == PIECE:
                st = stage_ref[slot]
            elif sub_off == 0:
                st = stage_ref[slot, :HALF, :]
            else:
                st = stage_ref[slot, HALF:, :]
            y = src_block + st
            ms = jnp.mean(y * y, axis=1, keepdims=True)
            inv = lax.rsqrt(ms + 1e-6)
            out_ref[pl.ds(row0, nrows), :] = y * inv * g_ref[:, :]

        pf_cw = {0: pf_m1}
        pf_ccw = {0: pf_p1}
        idx4 = (pos + N_HOP) % N_RING
        pf4 = None

        def mk_hop(i, comm, send_sems, recv_sems, tgt, src):
            if i == N_HOP - 1:
                dst = comm.at[i % 2, pl.ds(0, HALF), :] if tgt == "cw" \
                    else comm.at[i % 2, pl.ds(HALF, HALF), :]
            else:
                dst = comm.at[i % 2]
            dev = (right_x, right_y, my_z) if tgt == "cw" \
                else (left_x, left_y, my_z)
            return pltpu.make_async_remote_copy(
                src_ref=src,
                dst_ref=dst,
                send_sem=send_sems.at[i % 2],
                recv_sem=recv_sems.at[i % 2],
                device_id=dev,
                device_id_type=pl.DeviceIdType.MESH,
            )

        def mk_cw(i, src):
            return mk_hop(i, b_comm, b_send, b_recv, "cw", src)

        def mk_ccw(i, src):
            return mk_hop(i, c_comm, c_send, c_recv, "ccw", src)

        cw = [cw0] + [None] * (N_HOP - 1)
        ccw = [ccw0] + [None] * (N_HOP - 1)

        for h in range(N_HOP):
            if h == 0:
                for d in (*cw0, *ccw0):
                    d.wait_recv()
                for d in (*cw0, *ccw0):
                    d.wait_send()
            else:
                cw[h].wait_recv()
                ccw[h].wait_recv()
                cw[h].wait_send()
                ccw[h].wait_send()

            if 1 <= h <= N_HOP - 2:
                pl.semaphore_signal(
                    b_credit, inc=1,
                    device_id=(left_x, left_y, my_z),
                    device_id_type=pl.DeviceIdType.MESH,
                )
                pl.semaphore_signal(
                    c_credit, inc=1,
                    device_id=(right_x, right_y, my_z),
                    device_id_type=pl.DeviceIdType.MESH,
                )

            if h + 1 < N_HOP:
                if h + 1 >= 2:
                    pl.semaphore_wait(b_credit, 1)
                cw_src = (
                    b_comm.at[h % 2] if h + 1 < N_HOP - 1
                    else b_comm.at[h % 2, pl.ds(0, HALF), :]
                )
                cw[h + 1] = mk_cw(h + 1, cw_src)
                cw[h + 1].start()
                if h + 1 >= 2:
                    pl.semaphore_wait(c_credit, 1)
                ccw_src = (
                    c_comm.at[h % 2] if h + 1 < N_HOP - 1
                    else c_comm.at[h % 2, pl.ds(HALF, HALF), :]
                )
                ccw[h + 1] = mk_ccw(h + 1, ccw_src)
                ccw[h + 1].start()

            if h < N_HOP - 1:
                ln_from(pf_cw[h], ((pos - h - 1) % N_RING) * PIECE,
                        b_comm[h % 2])
                ln_from(pf_ccw[h], ((pos + h + 1) % N_RING) * PIECE,
                        c_comm[h % 2])
                if h == 0:
                    ln_from(pf_own, my_off, out_ref[pl.ds(my_off, PIECE), :])
                if h + 2 < N_HOP:
                    pf_cw[h + 1] = prefetch_piece((pos - h - 2) % N_RING)
                    pf_ccw[h + 1] = prefetch_piece((pos + h + 2) % N_RING)
                else:
                    pf4 = prefetch_piece(idx4)
            else:
                ln_from(pf4, idx4 * PIECE, b_comm[h % 2, :HALF, :],
                        nrows=HALF, sub_off=0)
                ln_from(pf4, idx4 * PIECE + HALF, c_comm[h % 2, HALF:, :],
                        nrows=HALF, sub_off=1, wait=False)

    return pl.pallas_call(
        body,
        out_shape=jax.ShapeDtypeStruct((M, D), jnp.float32),
        in_specs=[
            pl.BlockSpec(memory_space=pl.ANY),
            pl.BlockSpec(memory_space=pl.ANY),
            pl.BlockSpec(memory_space=pltpu.VMEM),
        ],
        out_specs=pl.BlockSpec(memory_space=pltpu.VMEM),
        scratch_shapes=[
            pltpu.VMEM((PIECE, D), jnp.float32),
            pltpu.VMEM((N_SUB, 2, QTR, D), jnp.float32),
            pltpu.VMEM((2, PIECE, D), jnp.float32),
            pltpu.VMEM((2, PIECE, D), jnp.float32),
            pltpu.VMEM((4, PIECE, D), jnp.float32),
            pltpu.SemaphoreType.DMA((N_SUB, 2)),
            pltpu.SemaphoreType.DMA((N_SUB, 2)),
            pltpu.SemaphoreType.DMA((3,)),
            pltpu.SemaphoreType.DMA((3,)),
            pltpu.SemaphoreType.DMA((3,)),
            pltpu.SemaphoreType.DMA((3,)),
            pltpu.SemaphoreType.REGULAR,
            pltpu.SemaphoreType.REGULAR,
            pltpu.SemaphoreType.DMA,
            pltpu.SemaphoreType.DMA((4,)),
        ],
        compiler_params=pltpu.CompilerParams(
            vmem_limit_bytes=100 * 1024 * 1024,
            collective_id=0,
        ),
    )(x, resid, g)


# device time: 146290 ns/iter; 2.2747x vs baseline; 1.0375x over previous
import jax
import jax.numpy as jnp
from jax import lax
from jax.experimental import pallas as pl
from jax.experimental.pallas import tpu as pltpu

N_Z = 4
N_RING = 8
M = 2048
D = 2048
PIECE = M // N_RING
HALF = PIECE // 2
N_SUB = 4
QTR = PIECE // N_SUB
N_HOP = 4


def _ring_coords(p):
    px = (p >= 4).astype(jnp.int32)
    py = jnp.where(p < 4, p, 7 - p)
    return px, py


def kernel(partial, resid, gamma):
    x = partial.reshape(M, D)
    g = gamma.reshape(1, D)

    def body(x_hbm, resid_hbm, g_ref, out_ref,
             acc_ref, a_comm, b_comm, c_comm, stage_ref,
             a_send, a_recv, b_send, b_recv, c_send, c_recv,
             b_credit, c_credit, local_sem, stage_sems):
        my_x = lax.axis_index("x")
        my_y = lax.axis_index("y")
        my_z = lax.axis_index("z")

        pos = jnp.where(my_x == 0, my_y, 7 - my_y)
        right_x, right_y = _ring_coords((pos + 1) % N_RING)
        left_x, left_y = _ring_coords((pos + 7) % N_RING)

        my_off = pos * PIECE

        is_edge = jnp.logical_or(my_z == 0, my_z == 3)
        partner = my_z ^ 1
        other_mid = 3 - my_z

        bar = pltpu.get_barrier_semaphore()
        for did in (
            (my_x, my_y, partner),
            (left_x, left_y, my_z),
            (right_x, right_y, my_z),
        ):
            pl.semaphore_signal(
                bar, inc=1, device_id=did,
                device_id_type=pl.DeviceIdType.MESH,
            )

        @pl.when(jnp.logical_not(is_edge))
        def _():
            pl.semaphore_signal(
                bar, inc=1, device_id=(my_x, my_y, other_mid),
                device_id_type=pl.DeviceIdType.MESH,
            )

        pl.semaphore_wait(bar, 3)

        @pl.when(jnp.logical_not(is_edge))
        def _():
            pl.semaphore_wait(bar, 1)

        cp = pltpu.make_async_copy(
            x_hbm.at[pl.ds(my_off, PIECE), :], acc_ref, local_sem
        )
        cp.start()

        pf_ctr = [0]

        def prefetch_piece(idx):
            slot = pf_ctr[0] % 4
            pf_ctr[0] += 1
            dma = pltpu.make_async_copy(
                resid_hbm.at[pl.ds(idx * PIECE, PIECE), :],
                stage_ref.at[slot],
                stage_sems.at[slot],
            )
            dma.start()
            return (dma, slot)

        pf_m1 = prefetch_piece((pos - 1) % N_RING)
        pf_p1 = prefetch_piece((pos + 1) % N_RING)
        pf_own = prefetch_piece(pos)

        cp.wait()

        def a_rdma(sub, slot, src, dst_z):
            return pltpu.make_async_remote_copy(
                src_ref=src,
                dst_ref=a_comm.at[sub, slot],
                send_sem=a_send.at[sub, slot],
                recv_sem=a_recv.at[sub, slot],
                device_id=(my_x, my_y, dst_z),
                device_id_type=pl.DeviceIdType.MESH,
            )

        def qtr(sub, ref=None, base=0):
            r = acc_ref if ref is None else ref
            return r.at[pl.ds(base + sub * QTR, QTR), :]

        def half(sub, ref=None, base=0):
            r = acc_ref if ref is None else ref
            return r.at[pl.ds(base + sub * HALF, HALF), :]

        def mk_fwd(dirn, h, j):
            comm, ss, rs = (
                (b_comm, b_send, b_recv) if dirn == "cw"
                else (c_comm, c_send, c_recv)
            )
            dev = (
                (right_x, right_y, my_z) if dirn == "cw"
                else (left_x, left_y, my_z)
            )
            if h == 0:
                src = half(j, out_ref, my_off)
            else:
                src = comm.at[(h - 1) % 2, pl.ds(j * HALF, HALF), :]
            return pltpu.make_async_remote_copy(
                src_ref=src,
                dst_ref=comm.at[h % 2, pl.ds(j * HALF, HALF), :],
                send_sem=ss.at[h % 2, j],
                recv_sem=rs.at[h % 2, j],
                device_id=dev,
                device_id_type=pl.DeviceIdType.MESH,
            )

        cw0 = [mk_fwd("cw", 0, j) for j in range(2)]
        ccw0 = [mk_fwd("ccw", 0, j) for j in range(2)]

        @pl.when(is_edge)
        def _():
            rd = [a_rdma(s, 0, qtr(s), partner) for s in range(N_SUB)]
            for s in range(N_SUB):
                rd[s].start()
            for s in range(N_SUB):
                fin = a_rdma(s, 1, qtr(s), partner)
                fin.wait_recv()
                out_ref[pl.ds(my_off + s * QTR, QTR), :] = a_comm[s, 1]
                if s % 2 == 1:
                    cw0[s // 2].start()
                    ccw0[s // 2].start()
            for s in range(N_SUB):
                rd[s].wait_send()

        @pl.when(jnp.logical_not(is_edge))
        def _():
            rcv = [a_rdma(s, 0, qtr(s), partner) for s in range(N_SUB)]
            ex = [None] * N_SUB
            for s in range(N_SUB):
                rcv[s].wait_recv()
                acc_ref[pl.ds(s * QTR, QTR), :] = (
                    acc_ref[pl.ds(s * QTR, QTR), :] + a_comm[s, 0]
                )
                ex[s] = a_rdma(s, 1, qtr(s), other_mid)
                ex[s].start()
            fin = [None] * N_SUB
            for s in range(N_SUB):
                ex[s].wait()
                out_ref[pl.ds(my_off + s * QTR, QTR), :] = (
                    acc_ref[pl.ds(s * QTR, QTR), :] + a_comm[s, 1]
                )
                fin[s] = a_rdma(
                    s, 1, qtr(s, out_ref, my_off), partner
                )
                fin[s].start()
                if s % 2 == 1:
                    cw0[s // 2].start()
                    ccw0[s // 2].start()
            for s in range(N_SUB):
                fin[s].wait_send()

        def ln_from(pf, row0, src_block, nrows=PIECE, sub_off=0, wait=True):
            dma, slot = pf
            if wait:
                dma.wait()
            if nrows == PIECE:
                st = stage_ref[slot]
            elif sub_off == 0:
                st = stage_ref[slot, :HALF, :]
            else:
                st = stage_ref[slot, HALF:, :]
            y = src_block + st
            ms = jnp.mean(y * y, axis=1, keepdims=True)
            inv = lax.rsqrt(ms + 1e-6)
            out_ref[pl.ds(row0, nrows), :] = y * inv * g_ref[:, :]

        pf_cw = {0: pf_m1}
        pf_ccw = {0: pf_p1}
        idx4 = (pos + N_HOP) % N_RING
        pf4 = None

        DD = {"cw": {(0, 0): cw0[0], (0, 1): cw0[1]},
              "ccw": {(0, 0): ccw0[0], (0, 1): ccw0[1]}}
        HOP3_J = {"cw": 0, "ccw": 1}
        CRED = {"cw": (b_credit, (left_x, left_y, my_z)),
                "ccw": (c_credit, (right_x, right_y, my_z))}

        for h in range(N_HOP):
            for j in range(2):
                for dirn in ("cw", "ccw"):
                    if h == N_HOP - 1 and j != HOP3_J[dirn]:
                        continue
                    dd = DD[dirn][(h, j)]
                    dd.wait_recv()
                    dd.wait_send()
                    sem, up_dev = CRED[dirn]
                    if h == 1 or (h == 2 and j == HOP3_J[dirn]):
                        pl.semaphore_signal(
                            sem, inc=1, device_id=up_dev,
                            device_id_type=pl.DeviceIdType.MESH,
                        )
                    nxt = h + 1
                    if nxt < N_HOP and (
                        nxt < N_HOP - 1 or j == HOP3_J[dirn]
                    ):
                        if nxt >= 2:
                            pl.semaphore_wait(sem, 1)
                        nd = mk_fwd(dirn, nxt, j)
                        DD[dirn][(nxt, j)] = nd
                        nd.start()

            if h < N_HOP - 1:
                ln_from(pf_cw[h], ((pos - h - 1) % N_RING) * PIECE,
                        b_comm[h % 2])
                ln_from(pf_ccw[h], ((pos + h + 1) % N_RING) * PIECE,
                        c_comm[h % 2])
                if h == 0:
                    ln_from(pf_own, my_off, out_ref[pl.ds(my_off, PIECE), :])
                if h + 2 < N_HOP:
                    pf_cw[h + 1] = prefetch_piece((pos - h - 2) % N_RING)
                    pf_ccw[h + 1] = prefetch_piece((pos + h + 2) % N_RING)
                else:
                    pf4 = prefetch_piece(idx4)
            else:
                ln_from(pf4, idx4 * PIECE, b_comm[h % 2, :HALF, :],
                        nrows=HALF, sub_off=0)
                ln_from(pf4, idx4 * PIECE + HALF, c_comm[h % 2, HALF:, :],
                        nrows=HALF, sub_off=1, wait=False)

    return pl.pallas_call(
        body,
        out_shape=jax.ShapeDtypeStruct((M, D), jnp.float32),
        in_specs=[
            pl.BlockSpec(memory_space=pl.ANY),
            pl.BlockSpec(memory_space=pl.ANY),
            pl.BlockSpec(memory_space=pltpu.VMEM),
        ],
        out_specs=pl.BlockSpec(memory_space=pltpu.VMEM),
        scratch_shapes=[
            pltpu.VMEM((PIECE, D), jnp.float32),
            pltpu.VMEM((N_SUB, 2, QTR, D), jnp.float32),
            pltpu.VMEM((2, PIECE, D), jnp.float32),
            pltpu.VMEM((2, PIECE, D), jnp.float32),
            pltpu.VMEM((4, PIECE, D), jnp.float32),
            pltpu.SemaphoreType.DMA((N_SUB, 2)),
            pltpu.SemaphoreType.DMA((N_SUB, 2)),
            pltpu.SemaphoreType.DMA((2, 2)),
            pltpu.SemaphoreType.DMA((2, 2)),
            pltpu.SemaphoreType.DMA((2, 2)),
            pltpu.SemaphoreType.DMA((2, 2)),
            pltpu.SemaphoreType.REGULAR,
            pltpu.SemaphoreType.REGULAR,
            pltpu.SemaphoreType.DMA,
            pltpu.SemaphoreType.DMA((4,)),
        ],
        compiler_params=pltpu.CompilerParams(
            vmem_limit_bytes=100 * 1024 * 1024,
            collective_id=0,
        ),
    )(x, resid, g)
